# baseline (speedup 1.0000x reference)
import functools

import jax
import jax.numpy as jnp
from jax import lax
from jax.experimental import pallas as pl
from jax.experimental.pallas import tpu as pltpu
from jax.experimental.pallas import tpu_sc as plsc

B = 1024
CTX = 50
D = 32
V = 100000

NC = 2
NS = 16
NW = NC * NS
B_PER_W = B // NW
ROWS_PER_W = B_PER_W * CTX

_sc_mesh = plsc.VectorSubcoreMesh(core_axis_name="c", subcore_axis_name="s")


@functools.partial(
    pl.kernel,
    out_type=jax.ShapeDtypeStruct((B, D), jnp.float32),
    mesh=_sc_mesh,
    scratch_types=[
        pltpu.VMEM((ROWS_PER_W,), jnp.int32),
        pltpu.VMEM((ROWS_PER_W, D), jnp.float32),
        pltpu.VMEM((B_PER_W, D), jnp.float32),
        pltpu.SemaphoreType.DMA,
    ],
    compiler_params=pltpu.CompilerParams(use_tc_tiling_on_sc=False),
)
def _pool_sc(idx_hbm, table_hbm, out_hbm, idx_v, rows_v, pooled_v, sem):
    wid = lax.axis_index("s") * NC + lax.axis_index("c")
    base = wid * ROWS_PER_W
    pltpu.sync_copy(idx_hbm.at[pl.ds(base, ROWS_PER_W)], idx_v)
    pltpu.async_copy(table_hbm.at[idx_v], rows_v, sem).wait()

    inv = jnp.float32(1.0 / CTX)

    def body_b(b, carry):
        def body_c(c, acc):
            a0, a1 = acc
            r = b * CTX + c
            a0 = a0 + rows_v[r, pl.ds(0, 16)]
            a1 = a1 + rows_v[r, pl.ds(16, 16)]
            return (a0, a1)

        a0, a1 = lax.fori_loop(
            0, CTX, body_c,
            (jnp.zeros((16,), jnp.float32), jnp.zeros((16,), jnp.float32)),
        )
        pooled_v[b, pl.ds(0, 16)] = a0 * inv
        pooled_v[b, pl.ds(16, 16)] = a1 * inv
        return carry

    lax.fori_loop(0, B_PER_W, body_b, 0)
    pltpu.sync_copy(pooled_v, out_hbm.at[pl.ds(wid * B_PER_W, B_PER_W)])


VB = 2048
_N_VB = (V + VB - 1) // VB


def _matmul_tc(x_ref, w_ref, b_ref, o_ref):
    o_ref[...] = (
        jnp.dot(x_ref[...], w_ref[...], preferred_element_type=jnp.float32)
        + b_ref[...]
    )


@jax.jit
def kernel(inputs, emb_table, dense_W, dense_b):
    idx = inputs.reshape(-1).astype(jnp.int32)
    pooled = _pool_sc(idx, emb_table)
    logits = pl.pallas_call(
        _matmul_tc,
        grid=(_N_VB,),
        in_specs=[
            pl.BlockSpec((B, D), lambda j: (0, 0)),
            pl.BlockSpec((D, VB), lambda j: (0, j)),
            pl.BlockSpec((1, VB), lambda j: (0, j)),
        ],
        out_specs=pl.BlockSpec((B, VB), lambda j: (0, j)),
        out_shape=jax.ShapeDtypeStruct((B, V), jnp.float32),
    )(pooled, dense_W, dense_b[None, :])
    return logits


# speedup vs baseline: 2.6892x; 2.6892x over previous
import functools

import jax
import jax.numpy as jnp
from jax import lax
from jax.experimental import pallas as pl
from jax.experimental.pallas import tpu as pltpu
from jax.experimental.pallas import tpu_sc as plsc

B = 1024
CTX = 50
D = 32
V = 100000

NC = 2
NS = 16
NW = NC * NS
B_PER_W = B // NW
ROWS_PER_W = B_PER_W * CTX

_sc_mesh = plsc.VectorSubcoreMesh(core_axis_name="c", subcore_axis_name="s")


@functools.partial(
    pl.kernel,
    out_type=jax.ShapeDtypeStruct((B, D), jnp.float32),
    mesh=_sc_mesh,
    scratch_types=[
        pltpu.VMEM((ROWS_PER_W,), jnp.int32),
        pltpu.VMEM((ROWS_PER_W, D), jnp.float32),
        pltpu.VMEM((B_PER_W, D), jnp.float32),
        pltpu.SemaphoreType.DMA,
    ],
    compiler_params=pltpu.CompilerParams(use_tc_tiling_on_sc=False),
)
def _pool_sc(idx_hbm, table_hbm, out_hbm, idx_v, rows_v, pooled_v, sem):
    wid = lax.axis_index("s") * NC + lax.axis_index("c")
    base = wid * ROWS_PER_W
    pltpu.sync_copy(idx_hbm.at[pl.ds(base, ROWS_PER_W)], idx_v)
    pltpu.async_copy(table_hbm.at[idx_v], rows_v, sem).wait()

    inv = jnp.float32(1.0 / CTX)

    def body_b(b, carry):
        def body_c(c, acc):
            a0, a1 = acc
            r = b * CTX + c
            a0 = a0 + rows_v[r, pl.ds(0, 16)]
            a1 = a1 + rows_v[r, pl.ds(16, 16)]
            return (a0, a1)

        a0, a1 = lax.fori_loop(
            0, CTX, body_c,
            (jnp.zeros((16,), jnp.float32), jnp.zeros((16,), jnp.float32)),
        )
        pooled_v[b, pl.ds(0, 16)] = a0 * inv
        pooled_v[b, pl.ds(16, 16)] = a1 * inv
        return carry

    lax.fori_loop(0, B_PER_W, body_b, 0)
    pltpu.sync_copy(pooled_v, out_hbm.at[pl.ds(wid * B_PER_W, B_PER_W)])


VB = 2048
_N_VB = (V + VB - 1) // VB


def _matmul_tc(w_ref, x_ref, b_ref, o_ref):
    dgn = (((0,), (0,)), ((), ()))
    o_ref[...] = jax.lax.dot_general(
        w_ref[...], x_ref[...], dgn, preferred_element_type=jnp.float32
    ) + jax.lax.dot_general(
        b_ref[...], jnp.ones((1, B), jnp.float32), dgn,
        preferred_element_type=jnp.float32,
    )


@jax.jit
def kernel(inputs, emb_table, dense_W, dense_b):
    idx = inputs.reshape(-1).astype(jnp.int32)
    pooled = _pool_sc(idx, emb_table)
    logits_t = pl.pallas_call(
        _matmul_tc,
        grid=(_N_VB,),
        in_specs=[
            pl.BlockSpec((D, VB), lambda j: (0, j)),
            pl.BlockSpec((D, B), lambda j: (0, 0)),
            pl.BlockSpec((1, VB), lambda j: (0, j)),
        ],
        out_specs=pl.BlockSpec((VB, B), lambda j: (j, 0)),
        out_shape=jax.ShapeDtypeStruct((V, B), jnp.float32),
    )(dense_W, pooled.T, dense_b[None, :])
    return logits_t.T
